# baseline (device time: 37769 ns/iter reference)
import jax
import jax.numpy as jnp
from jax import lax
from jax.experimental import pallas as pl
from jax.experimental.pallas import tpu as pltpu

N_DEV = 8
S = 2


def _gelu(z):
    return 0.5 * z * (1.0 + jnp.tanh(0.7978845608 * (z + 0.044715 * z * z * z)))


def kernel(A, B):
    m, k = A.shape
    k2, n = B.shape
    rows = m // N_DEV
    srows = rows // S

    def body(a_ref, b_ref, out_ref,
             src_ref, rs_ref, ag_ref,
             rs_send_sems, rs_recv_sems, ag_send_sems, ag_recv_sems):
        my_pos = lax.axis_index("i")

        barrier_sem = pltpu.get_barrier_semaphore()
        for d in range(1, N_DEV):
            peer = lax.rem(my_pos + d, N_DEV)
            pl.semaphore_signal(
                barrier_sem, inc=1,
                device_id=(peer,), device_id_type=pl.DeviceIdType.MESH,
            )

        b_bf = b_ref[:, :].astype(jnp.bfloat16)

        def block(j):
            a_blk = a_ref[pl.ds(j * rows, rows), :].astype(jnp.bfloat16)
            return jnp.dot(a_blk, b_bf, preferred_element_type=jnp.float32)

        send_rdmas = []

        for d in range(1, N_DEV):
            j = lax.rem(my_pos + d, N_DEV)
            blk = block(j)
            for s in range(S):
                src_ref[s, j, :, :] = blk[s * srows:(s + 1) * srows, :].astype(
                    jnp.bfloat16)
            if d == 1:
                pl.semaphore_wait(barrier_sem, N_DEV - 1)
            for s in range(S):
                rdma = pltpu.make_async_remote_copy(
                    src_ref=src_ref.at[s, j],
                    dst_ref=rs_ref.at[s, my_pos],
                    send_sem=rs_send_sems.at[s, j],
                    recv_sem=rs_recv_sems.at[s, my_pos],
                    device_id=(j,),
                    device_id_type=pl.DeviceIdType.MESH,
                )
                rdma.start()
                send_rdmas.append(rdma)

        own = block(my_pos)
        for s in range(S):
            rs_ref[s, my_pos, :, :] = own[s * srows:(s + 1) * srows, :].astype(
                jnp.bfloat16)

        def rs_descriptor(s, i):
            return pltpu.make_async_remote_copy(
                src_ref=src_ref.at[s, i],
                dst_ref=rs_ref.at[s, i],
                send_sem=rs_send_sems.at[s, i],
                recv_sem=rs_recv_sems.at[s, i],
                device_id=(i,),
                device_id_type=pl.DeviceIdType.MESH,
            )

        def ag_descriptor(s, i):
            return pltpu.make_async_remote_copy(
                src_ref=ag_ref.at[s, i],
                dst_ref=ag_ref.at[s, i],
                send_sem=ag_send_sems.at[s, i],
                recv_sem=ag_recv_sems.at[s, i],
                device_id=(i,),
                device_id_type=pl.DeviceIdType.MESH,
            )

        for s in range(S):
            for d in range(1, N_DEV):
                i = lax.rem(my_pos + d, N_DEV)
                rs_descriptor(s, i).wait_recv()
            acc = rs_ref[s, 0, :, :].astype(jnp.float32)
            for i in range(1, N_DEV):
                acc += rs_ref[s, i, :, :].astype(jnp.float32)
            ag_ref[s, my_pos, :, :] = _gelu(acc).astype(jnp.bfloat16)
            for d in range(1, N_DEV):
                j = lax.rem(my_pos + d, N_DEV)
                rdma = pltpu.make_async_remote_copy(
                    src_ref=ag_ref.at[s, my_pos],
                    dst_ref=ag_ref.at[s, my_pos],
                    send_sem=ag_send_sems.at[s, j],
                    recv_sem=ag_recv_sems.at[s, my_pos],
                    device_id=(j,),
                    device_id_type=pl.DeviceIdType.MESH,
                )
                rdma.start()
                send_rdmas.append(rdma)

        for s in range(S):
            for d in range(1, N_DEV):
                i = lax.rem(my_pos + d, N_DEV)
                ag_descriptor(s, i).wait_recv()
            for i in range(N_DEV):
                r0 = i * rows + s * srows
                out_ref[r0:r0 + srows, :] = ag_ref[s, i, :, :].astype(
                    jnp.float32)

        for rdma in send_rdmas:
            rdma.wait_send()

    return pl.pallas_call(
        body,
        out_shape=jax.ShapeDtypeStruct((m, n), jnp.float32),
        in_specs=[
            pl.BlockSpec(memory_space=pltpu.VMEM),
            pl.BlockSpec(memory_space=pltpu.VMEM),
        ],
        out_specs=pl.BlockSpec(memory_space=pltpu.VMEM),
        scratch_shapes=[
            pltpu.VMEM((S, N_DEV, srows, n), jnp.bfloat16),
            pltpu.VMEM((S, N_DEV, srows, n), jnp.bfloat16),
            pltpu.VMEM((S, N_DEV, srows, n), jnp.bfloat16),
            pltpu.SemaphoreType.DMA((S, N_DEV)),
            pltpu.SemaphoreType.DMA((S, N_DEV)),
            pltpu.SemaphoreType.DMA((S, N_DEV)),
            pltpu.SemaphoreType.DMA((S, N_DEV)),
        ],
        compiler_params=pltpu.CompilerParams(collective_id=0),
    )(A, B)


# device time: 34713 ns/iter; 1.0880x vs baseline; 1.0880x over previous
import jax
import jax.numpy as jnp
from jax import lax
from jax.experimental import pallas as pl
from jax.experimental.pallas import tpu as pltpu

N_DEV = 8
S = 2


def _gelu(z):
    return 0.5 * z * (1.0 + jnp.tanh(0.7978845608 * (z + 0.044715 * z * z * z)))


def kernel(A, B):
    m, k = A.shape
    k2, n = B.shape
    rows = m // N_DEV
    srows = rows // S

    def body(a_ref, b_ref, out_ref,
             src_ref, rs_ref, ag_ref,
             rs_send_sems, rs_recv_sems, ag_send_sems, ag_recv_sems):
        my_pos = lax.axis_index("i")

        barrier_sem = pltpu.get_barrier_semaphore()
        for d in range(1, N_DEV):
            peer = lax.rem(my_pos + d, N_DEV)
            pl.semaphore_signal(
                barrier_sem, inc=1,
                device_id=(peer,), device_id_type=pl.DeviceIdType.MESH,
            )

        partial = jnp.dot(
            a_ref[:, :].astype(jnp.bfloat16),
            b_ref[:, :].astype(jnp.bfloat16),
            preferred_element_type=jnp.float32,
        )
        for s in range(S):
            for j in range(N_DEV):
                r0 = j * rows + s * srows
                src_ref[s, j, :, :] = partial[r0:r0 + srows, :].astype(
                    jnp.bfloat16)

        pl.semaphore_wait(barrier_sem, N_DEV - 1)

        send_rdmas = []

        def rs_descriptor(s, i):
            return pltpu.make_async_remote_copy(
                src_ref=src_ref.at[s, i],
                dst_ref=rs_ref.at[s, i],
                send_sem=rs_send_sems.at[s, i],
                recv_sem=rs_recv_sems.at[s, i],
                device_id=(i,),
                device_id_type=pl.DeviceIdType.MESH,
            )

        def ag_descriptor(s, i):
            return pltpu.make_async_remote_copy(
                src_ref=ag_ref.at[s, i],
                dst_ref=ag_ref.at[s, i],
                send_sem=ag_send_sems.at[s, i],
                recv_sem=ag_recv_sems.at[s, i],
                device_id=(i,),
                device_id_type=pl.DeviceIdType.MESH,
            )

        for s in range(S):
            for d in range(1, N_DEV):
                j = lax.rem(my_pos + d, N_DEV)
                rdma = pltpu.make_async_remote_copy(
                    src_ref=src_ref.at[s, j],
                    dst_ref=rs_ref.at[s, my_pos],
                    send_sem=rs_send_sems.at[s, j],
                    recv_sem=rs_recv_sems.at[s, my_pos],
                    device_id=(j,),
                    device_id_type=pl.DeviceIdType.MESH,
                )
                rdma.start()
                send_rdmas.append(rdma)
            rs_ref[s, my_pos, :, :] = src_ref[s, my_pos, :, :]

        for s in range(S):
            for d in range(1, N_DEV):
                i = lax.rem(my_pos + d, N_DEV)
                rs_descriptor(s, i).wait_recv()
            terms = [rs_ref[s, i, :, :].astype(jnp.float32)
                     for i in range(N_DEV)]
            while len(terms) > 1:
                terms = [terms[p] + terms[p + 1]
                         for p in range(0, len(terms), 2)]
            g = _gelu(terms[0]).astype(jnp.bfloat16)
            ag_ref[s, my_pos, :, :] = g
            for d in range(1, N_DEV):
                j = lax.rem(my_pos + d, N_DEV)
                rdma = pltpu.make_async_remote_copy(
                    src_ref=ag_ref.at[s, my_pos],
                    dst_ref=ag_ref.at[s, my_pos],
                    send_sem=ag_send_sems.at[s, j],
                    recv_sem=ag_recv_sems.at[s, my_pos],
                    device_id=(j,),
                    device_id_type=pl.DeviceIdType.MESH,
                )
                rdma.start()
                send_rdmas.append(rdma)
            r0 = my_pos * rows + s * srows
            out_ref[pl.ds(r0, srows), :] = g.astype(jnp.float32)

        for s in range(S):
            for d in range(1, N_DEV):
                i = lax.rem(my_pos + d, N_DEV)
                ag_descriptor(s, i).wait_recv()
                r0 = i * rows + s * srows
                out_ref[pl.ds(r0, srows), :] = ag_ref[s, i, :, :].astype(
                    jnp.float32)

        for rdma in send_rdmas:
            rdma.wait_send()

    return pl.pallas_call(
        body,
        out_shape=jax.ShapeDtypeStruct((m, n), jnp.float32),
        in_specs=[
            pl.BlockSpec(memory_space=pltpu.VMEM),
            pl.BlockSpec(memory_space=pltpu.VMEM),
        ],
        out_specs=pl.BlockSpec(memory_space=pltpu.VMEM),
        scratch_shapes=[
            pltpu.VMEM((S, N_DEV, srows, n), jnp.bfloat16),
            pltpu.VMEM((S, N_DEV, srows, n), jnp.bfloat16),
            pltpu.VMEM((S, N_DEV, srows, n), jnp.bfloat16),
            pltpu.SemaphoreType.DMA((S, N_DEV)),
            pltpu.SemaphoreType.DMA((S, N_DEV)),
            pltpu.SemaphoreType.DMA((S, N_DEV)),
            pltpu.SemaphoreType.DMA((S, N_DEV)),
        ],
        compiler_params=pltpu.CompilerParams(collective_id=0),
    )(A, B)


# device time: 33906 ns/iter; 1.1139x vs baseline; 1.0238x over previous
import jax
import jax.numpy as jnp
from jax import lax
from jax.experimental import pallas as pl
from jax.experimental.pallas import tpu as pltpu

N_DEV = 8
S = 2


def _gelu(z):
    return 0.5 * z * (1.0 + jnp.tanh(0.7978845608 * (z + 0.044715 * z * z * z)))


def kernel(A, B):
    m, k = A.shape
    k2, n = B.shape
    rows = m // N_DEV
    srows = rows // S

    def body(a_ref, b_ref, out_ref,
             src_ref, rs_ref,
             rs_send_sems, rs_recv_sems, ag_send_sems, ag_recv_sems):
        my_pos = lax.axis_index("i")

        barrier_sem = pltpu.get_barrier_semaphore()
        for d in range(1, N_DEV):
            peer = lax.rem(my_pos + d, N_DEV)
            pl.semaphore_signal(
                barrier_sem, inc=1,
                device_id=(peer,), device_id_type=pl.DeviceIdType.MESH,
            )

        partial = jnp.dot(
            a_ref[:, :].astype(jnp.bfloat16),
            b_ref[:, :].astype(jnp.bfloat16),
            preferred_element_type=jnp.float32,
        ).astype(jnp.bfloat16)
        for s in range(S):
            for j in range(N_DEV):
                r0 = j * rows + s * srows
                src_ref[s, j, :, :] = partial[r0:r0 + srows, :]

        pl.semaphore_wait(barrier_sem, N_DEV - 1)

        send_rdmas = []

        def rs_descriptor(s, i):
            return pltpu.make_async_remote_copy(
                src_ref=src_ref.at[s, i],
                dst_ref=rs_ref.at[s, i],
                send_sem=rs_send_sems.at[s, i],
                recv_sem=rs_recv_sems.at[s, i],
                device_id=(i,),
                device_id_type=pl.DeviceIdType.MESH,
            )

        def ag_descriptor(s, i):
            r0 = i * rows + s * srows
            return pltpu.make_async_remote_copy(
                src_ref=out_ref.at[pl.ds(r0, srows)],
                dst_ref=out_ref.at[pl.ds(r0, srows)],
                send_sem=ag_send_sems.at[s, i],
                recv_sem=ag_recv_sems.at[s, i],
                device_id=(i,),
                device_id_type=pl.DeviceIdType.MESH,
            )

        for s in range(S):
            for d in range(1, N_DEV):
                j = lax.rem(my_pos + d, N_DEV)
                rdma = pltpu.make_async_remote_copy(
                    src_ref=src_ref.at[s, j],
                    dst_ref=rs_ref.at[s, my_pos],
                    send_sem=rs_send_sems.at[s, j],
                    recv_sem=rs_recv_sems.at[s, my_pos],
                    device_id=(j,),
                    device_id_type=pl.DeviceIdType.MESH,
                )
                rdma.start()
                send_rdmas.append(rdma)
            rs_ref[s, my_pos, :, :] = src_ref[s, my_pos, :, :]

        for s in range(S):
            for d in range(1, N_DEV):
                i = lax.rem(my_pos + d, N_DEV)
                rs_descriptor(s, i).wait_recv()
            terms = [rs_ref[s, i, :, :].astype(jnp.float32)
                     for i in range(N_DEV)]
            while len(terms) > 1:
                terms = [terms[p] + terms[p + 1]
                         for p in range(0, len(terms), 2)]
            my_r0 = my_pos * rows + s * srows
            out_ref[pl.ds(my_r0, srows), :] = _gelu(terms[0]).astype(
                jnp.bfloat16)
            for d in range(1, N_DEV):
                j = lax.rem(my_pos + d, N_DEV)
                rdma = pltpu.make_async_remote_copy(
                    src_ref=out_ref.at[pl.ds(my_r0, srows)],
                    dst_ref=out_ref.at[pl.ds(my_r0, srows)],
                    send_sem=ag_send_sems.at[s, j],
                    recv_sem=ag_recv_sems.at[s, my_pos],
                    device_id=(j,),
                    device_id_type=pl.DeviceIdType.MESH,
                )
                rdma.start()
                send_rdmas.append(rdma)

        for s in range(S):
            for d in range(1, N_DEV):
                i = lax.rem(my_pos + d, N_DEV)
                ag_descriptor(s, i).wait_recv()

        for rdma in send_rdmas:
            rdma.wait_send()

    return pl.pallas_call(
        body,
        out_shape=jax.ShapeDtypeStruct((m, n), jnp.bfloat16),
        in_specs=[
            pl.BlockSpec(memory_space=pltpu.VMEM),
            pl.BlockSpec(memory_space=pltpu.VMEM),
        ],
        out_specs=pl.BlockSpec(memory_space=pltpu.VMEM),
        scratch_shapes=[
            pltpu.VMEM((S, N_DEV, srows, n), jnp.bfloat16),
            pltpu.VMEM((S, N_DEV, srows, n), jnp.bfloat16),
            pltpu.SemaphoreType.DMA((S, N_DEV)),
            pltpu.SemaphoreType.DMA((S, N_DEV)),
            pltpu.SemaphoreType.DMA((S, N_DEV)),
            pltpu.SemaphoreType.DMA((S, N_DEV)),
        ],
        compiler_params=pltpu.CompilerParams(collective_id=0),
    )(A, B)
